# baseline (speedup 1.0000x reference)
import functools

import jax
import jax.numpy as jnp
from jax import lax
from jax.experimental import pallas as pl
from jax.experimental.pallas import tpu as pltpu
from jax.experimental.pallas import tpu_sc as plsc

_NUM_CORES = 2
_NUM_SUBCORES = 16
_NW = _NUM_CORES * _NUM_SUBCORES
_CHUNK = 8
_LANES = 128
_VREG = 16
_NBUF = 4


def _make_gather(total_q: int, per_w: int, n_chunk: int, b: int, d: int,
                 tail: int):
    mesh = plsc.VectorSubcoreMesh(core_axis_name="c", subcore_axis_name="s")
    pieces = d // _LANES
    qchunk = _CHUNK * pieces
    n_group = n_chunk // _NBUF

    @functools.partial(
        pl.kernel,
        mesh=mesh,
        out_type=jax.ShapeDtypeStruct((total_q, _LANES), jnp.float32),
        scratch_types=[
            pltpu.VMEM((n_chunk, _CHUNK), jnp.int32),
            pltpu.VMEM((_CHUNK, d), jnp.float32),
            pltpu.VMEM((_CHUNK, d), jnp.float32),
            pltpu.VMEM((_CHUNK, d), jnp.float32),
            pltpu.VMEM((_CHUNK, d), jnp.float32),
            pltpu.VMEM((qchunk, _LANES), jnp.float32),
            pltpu.VMEM((qchunk, _LANES), jnp.float32),
            pltpu.SemaphoreType.DMA,
            pltpu.SemaphoreType.DMA,
            pltpu.SemaphoreType.DMA,
            pltpu.SemaphoreType.DMA,
            pltpu.SemaphoreType.DMA,
            pltpu.SemaphoreType.DMA,
        ],
    )
    def gather_kernel(table_hbm, gidx_hbm, out_hbm, gidx_v,
                      rows0, rows1, rows2, rows3, s0, s1,
                      gsem0, gsem1, gsem2, gsem3, osem0, osem1):
        wid = lax.axis_index("s") * _NUM_CORES + lax.axis_index("c")
        pltpu.sync_copy(gidx_hbm.at[wid], gidx_v)
        base_q = jnp.where(wid == _NW - 1, tail, wid * per_w) * pieces

        rows = (rows0, rows1, rows2, rows3)
        gsems = (gsem0, gsem1, gsem2, gsem3)
        svmem = (s0, s1)
        osems = (osem0, osem1)

        def gather(c, buf, sem):
            return pltpu.make_async_copy(table_hbm.at[gidx_v.at[c]], buf, sem)

        def out_write(c, buf, sem):
            return pltpu.make_async_copy(
                buf, out_hbm.at[pl.ds(base_q + c * qchunk, qchunk)], sem)

        def interleave(src, dst):
            for j in range(_CHUNK // b):
                for k in range(pieces):
                    for bi in range(b):
                        for v in range(_LANES // _VREG):
                            dst[(j * pieces + k) * b + bi,
                                pl.ds(v * _VREG, _VREG)] = (
                                src[j * b + bi,
                                    pl.ds(k * _LANES + v * _VREG, _VREG)])

        for s in range(_NBUF):
            gather(s, rows[s], gsems[s]).start()

        def group(m, carry):
            for s in range(_NBUF):
                c = m * _NBUF + s
                gather(c, rows[s], gsems[s]).wait()

                if s >= 2:
                    out_write(c - 2, svmem[s % 2], osems[s % 2]).wait()
                else:
                    @pl.when(m >= 1)
                    def _wait_prev(c=c, s=s):
                        out_write(c - 2, svmem[s % 2], osems[s % 2]).wait()

                interleave(rows[s], svmem[s % 2])
                out_write(c, svmem[s % 2], osems[s % 2]).start()

                @pl.when(m < n_group - 1)
                def _next_gather(c=c, s=s):
                    gather(c + _NBUF, rows[s], gsems[s]).start()
            return carry

        lax.fori_loop(0, n_group, group, 0)
        out_write(n_chunk - 2, s0, osem0).wait()
        out_write(n_chunk - 1, s1, osem1).wait()

    return gather_kernel


def kernel(last_hidden_state, offsets, mask):
    del mask
    b, t, d = last_hidden_state.shape
    n = offsets.shape[1]
    r = n - 2
    total_g = b * r
    pieces = d // _LANES
    total_q = total_g * pieces

    starts = offsets[:, 1 : n - 1, 0]
    src = (starts + (jnp.arange(b, dtype=jnp.int32) * t)[:, None]).T.reshape(-1)

    per_w = -(-total_g // _NW)
    per_w = -(-per_w // (_NBUF * _CHUNK)) * (_NBUF * _CHUNK)
    n_chunk = per_w // _CHUNK
    tail = total_g - per_w
    assert tail % 8 == 0 and tail % b == 0 and tail >= 0
    assert _CHUNK % b == 0 and per_w % b == 0
    gidx = jnp.concatenate(
        [src[: (_NW - 1) * per_w], src[tail:]]).reshape(_NW, n_chunk, _CHUNK)

    table = last_hidden_state.reshape(b * t, d)
    out = _make_gather(total_q, per_w, n_chunk, b, d, tail)(table, gidx)
    return (out.reshape(r, pieces, b, _LANES)
            .transpose(2, 0, 1, 3)
            .reshape(b, r, d))
